# baseline (speedup 1.0000x reference)
import functools

import jax
import jax.numpy as jnp
from jax import lax
from jax.experimental import pallas as pl
from jax.experimental.pallas import tpu as pltpu
from jax.experimental.pallas import tpu_sc as plsc

_EMBD = 32
_B = 16384 * 50
_PACK = 4
_PROW = 128

_NC = 2
_NS = 16
_NW = _NC * _NS
_BPW = _B // _NW
_C = 32
_N = _BPW // _C
_NBUF = 8
_LOOK = 4
_L = 16

_mesh = plsc.VectorSubcoreMesh(core_axis_name="c", subcore_axis_name="s")


@functools.partial(
    pl.kernel,
    mesh=_mesh,
    out_type=jax.ShapeDtypeStruct((_B * _EMBD,), jnp.float32),
    scratch_types=(
        [pltpu.VMEM((_BPW,), jnp.int32), pltpu.VMEM((_BPW,), jnp.int32)]
        + [pltpu.VMEM((_C, _PROW), jnp.float32) for _ in range(_NBUF)]
        + [pltpu.VMEM((_C * _EMBD,), jnp.float32) for _ in range(_NBUF)]
        + [pltpu.SemaphoreType.DMA for _ in range(2 * _NBUF)]
    ),
    compiler_params=pltpu.CompilerParams(needs_layout_passes=False),
)
def _gather_kernel(idx_hbm, table_hbm, out_hbm, idx_v, ridx_v, *rest):
    gbufs = rest[:_NBUF]
    obufs = rest[_NBUF : 2 * _NBUF]
    gsem = rest[2 * _NBUF : 3 * _NBUF]
    wsem = rest[3 * _NBUF :]

    wid = lax.axis_index("s") * _NC + lax.axis_index("c")
    base = wid * _BPW
    pltpu.sync_copy(idx_hbm.at[pl.ds(base, _BPW)], idx_v)

    @pl.loop(0, _BPW // _L)
    def _prep(i):
        off = i * _L
        e = idx_v[pl.ds(off, _L)]
        ridx_v[pl.ds(off, _L)] = e >> 2

    def start_gather(j, slot):
        pltpu.async_copy(
            table_hbm.at[ridx_v.at[pl.ds(j * _C, _C)]], gbufs[slot], gsem[slot]
        )

    def wait_gather(slot):
        pltpu.make_async_copy(
            table_hbm.at[ridx_v.at[pl.ds(0, _C)]], gbufs[slot], gsem[slot]
        ).wait()

    def extract(j, slot):
        gbuf, obuf = gbufs[slot], obufs[slot]
        for g in range(_C // _L):
            e = idx_v[pl.ds(j * _C + g * _L, _L)]
            seg = (e & 3) << 5
            rows = lax.iota(jnp.int32, _L) + g * _L
            dst0 = rows * _EMBD
            for c in range(_EMBD):
                v = plsc.load_gather(gbuf, [rows, seg + c])
                plsc.store_scatter(obuf, [dst0 + c], v)

    def start_write(j, slot):
        pltpu.async_copy(
            obufs[slot],
            out_hbm.at[pl.ds((base + j * _C) * _EMBD, _C * _EMBD)],
            wsem[slot],
        )

    def wait_write(slot):
        pltpu.make_async_copy(
            obufs[slot], out_hbm.at[pl.ds(0, _C * _EMBD)], wsem[slot]
        ).wait()

    for j in range(_LOOK):
        start_gather(j, j)

    for j in range(_LOOK):
        start_gather(j + _LOOK, j + _LOOK)
        wait_gather(j)
        extract(j, j)
        start_write(j, j)

    @pl.loop(_LOOK, _N - _LOOK, step=_NBUF)
    def _steady(g):
        for b in range(_NBUF):
            j = g + b
            s_ahead = (_LOOK + b + _LOOK) % _NBUF
            wait_write(s_ahead)
            start_gather(j + _LOOK, s_ahead)
            slot = (_LOOK + b) % _NBUF
            wait_gather(slot)
            extract(j, slot)
            start_write(j, slot)

    for t in range(_LOOK):
        j = _N - _LOOK + t
        slot = j % _NBUF
        wait_gather(slot)
        extract(j, slot)
        start_write(j, slot)

    for b in range(_NBUF):
        wait_write(b)


def kernel(inputs, context_weight):
    idx = inputs.reshape(-1).astype(jnp.int32)
    table128 = context_weight.reshape(-1, _PROW)
    out = _gather_kernel(idx, table128)
    return out.reshape(inputs.shape[0], inputs.shape[1], _EMBD)


# speedup vs baseline: 1.7115x; 1.7115x over previous
import functools

import jax
import jax.numpy as jnp
from jax import lax
from jax.experimental import pallas as pl
from jax.experimental.pallas import tpu as pltpu
from jax.experimental.pallas import tpu_sc as plsc

_EMBD = 32
_BATCH = 16384
_HIST = 50
_HPAD = 56
_PROW = 128

_NC = 2
_NS = 16
_NW = _NC * _NS
_BPW = _BATCH // _NW
_NBUF = 4
_LOOK = 2

_mesh = plsc.VectorSubcoreMesh(core_axis_name="c", subcore_axis_name="s")


@functools.partial(
    pl.kernel,
    mesh=_mesh,
    out_type=jax.ShapeDtypeStruct((_BATCH, _HIST, _EMBD), jnp.float32),
    scratch_types=(
        [
            pltpu.VMEM((_BPW * _HPAD,), jnp.int32),
            pltpu.VMEM((_BPW * _HPAD + 8,), jnp.int32),
        ]
        + [pltpu.VMEM((_HIST, _PROW), jnp.float32) for _ in range(_NBUF)]
        + [pltpu.VMEM((_HIST, _EMBD), jnp.float32) for _ in range(_NBUF)]
        + [pltpu.SemaphoreType.DMA for _ in range(2 * _NBUF)]
    ),
    compiler_params=pltpu.CompilerParams(needs_layout_passes=False),
)
def _gather_kernel(ridx_hbm, seg_hbm, table_hbm, out_hbm, ridx_v, seg_v, *rest):
    gbufs = rest[:_NBUF]
    obufs = rest[_NBUF : 2 * _NBUF]
    gsem = rest[2 * _NBUF : 3 * _NBUF]
    wsem = rest[3 * _NBUF :]

    wid = lax.axis_index("s") * _NC + lax.axis_index("c")
    base = wid * _BPW
    pltpu.sync_copy(ridx_hbm.at[pl.ds(base * _HPAD, _BPW * _HPAD)], ridx_v)
    pltpu.sync_copy(
        seg_hbm.at[pl.ds(base * _HPAD, _BPW * _HPAD)],
        seg_v.at[pl.ds(0, _BPW * _HPAD)],
    )

    def start_gather(j, slot):
        pltpu.async_copy(
            table_hbm.at[ridx_v.at[pl.ds(j * _HPAD, _HIST)]],
            gbufs[slot],
            gsem[slot],
        )

    def wait_gather(slot):
        pltpu.make_async_copy(
            table_hbm.at[ridx_v.at[pl.ds(0, _HIST)]], gbufs[slot], gsem[slot]
        ).wait()

    def extract(j, slot):
        gbuf, obuf = gbufs[slot], obufs[slot]
        for q in range(4):
            sv = seg_v[pl.ds(j * _HPAD + 16 * q, 16)]
            for l in range(16):
                h = 16 * q + l
                if h >= _HIST:
                    break
                s = sv[l]
                obuf[h, pl.ds(0, 16)] = gbuf[h, pl.ds(s, 16)]
                obuf[h, pl.ds(16, 16)] = gbuf[h, pl.ds(s + 16, 16)]

    def start_write(j, slot):
        pltpu.async_copy(obufs[slot], out_hbm.at[base + j], wsem[slot])

    def wait_write(slot):
        pltpu.make_async_copy(obufs[slot], out_hbm.at[0], wsem[slot]).wait()

    for j in range(_LOOK):
        start_gather(j, j)

    for j in range(_LOOK):
        start_gather(j + _LOOK, j + _LOOK)
        wait_gather(j)
        extract(j, j)
        start_write(j, j)

    @pl.loop(_LOOK, _BPW - _LOOK, step=_NBUF)
    def _steady(g):
        for b in range(_NBUF):
            j = g + b
            s_ahead = (_LOOK + b + _LOOK) % _NBUF
            wait_write(s_ahead)
            start_gather(j + _LOOK, s_ahead)
            slot = (_LOOK + b) % _NBUF
            wait_gather(slot)
            extract(j, slot)
            start_write(j, slot)

    for t in range(_LOOK):
        j = _BPW - _LOOK + t
        slot = j % _NBUF
        wait_gather(slot)
        extract(j, slot)
        start_write(j, slot)

    for b in range(_NBUF):
        wait_write(b)


def kernel(inputs, context_weight):
    idx = inputs.astype(jnp.int32)
    ridx = jnp.pad(idx >> 2, ((0, 0), (0, _HPAD - _HIST))).reshape(-1)
    seg = jnp.pad((idx & 3) << 5, ((0, 0), (0, _HPAD - _HIST))).reshape(-1)
    table128 = context_weight.reshape(-1, _PROW)
    return _gather_kernel(ridx, seg, table128)


# speedup vs baseline: 1.7116x; 1.0001x over previous
import functools

import jax
import jax.numpy as jnp
from jax import lax
from jax.experimental import pallas as pl
from jax.experimental.pallas import tpu as pltpu
from jax.experimental.pallas import tpu_sc as plsc

_EMBD = 32
_BATCH = 16384
_HIST = 50
_HPAD = 56
_PROW = 128

_NC = 2
_NS = 16
_NW = _NC * _NS
_BPW = _BATCH // _NW
_NBUF = 4
_LOOK = 2

_mesh = plsc.VectorSubcoreMesh(core_axis_name="c", subcore_axis_name="s")


@functools.partial(
    pl.kernel,
    mesh=_mesh,
    out_type=jax.ShapeDtypeStruct((_BATCH, _HIST, _EMBD), jnp.float32),
    scratch_types=(
        [
            pltpu.VMEM((_BPW * _HPAD,), jnp.int32),
            pltpu.VMEM((_BPW * _HPAD + 8,), jnp.int32),
        ]
        + [pltpu.VMEM((_HIST, _PROW), jnp.float32) for _ in range(_NBUF)]
        + [pltpu.VMEM((_HIST, _EMBD), jnp.float32) for _ in range(_NBUF)]
        + [pltpu.SemaphoreType.DMA for _ in range(2 * _NBUF)]
    ),
)
def _gather_kernel(ridx_hbm, seg_hbm, table_hbm, out_hbm, ridx_v, seg_v, *rest):
    gbufs = rest[:_NBUF]
    obufs = rest[_NBUF : 2 * _NBUF]
    gsem = rest[2 * _NBUF : 3 * _NBUF]
    wsem = rest[3 * _NBUF :]

    wid = lax.axis_index("s") * _NC + lax.axis_index("c")
    base = wid * _BPW
    pltpu.sync_copy(ridx_hbm.at[pl.ds(base * _HPAD, _BPW * _HPAD)], ridx_v)
    pltpu.sync_copy(
        seg_hbm.at[pl.ds(base * _HPAD, _BPW * _HPAD)],
        seg_v.at[pl.ds(0, _BPW * _HPAD)],
    )

    def start_gather(j, slot):
        pltpu.async_copy(
            table_hbm.at[ridx_v.at[pl.ds(j * _HPAD, _HIST)]],
            gbufs[slot],
            gsem[slot],
        )

    def wait_gather(slot):
        pltpu.make_async_copy(
            table_hbm.at[ridx_v.at[pl.ds(0, _HIST)]], gbufs[slot], gsem[slot]
        ).wait()

    def extract(j, slot):
        gbuf, obuf = gbufs[slot], obufs[slot]
        for q in range(4):
            sv = seg_v[pl.ds(j * _HPAD + 16 * q, 16)]
            for l in range(16):
                h = 16 * q + l
                if h >= _HIST:
                    break
                s = sv[l]
                obuf[h, pl.ds(0, 16)] = gbuf[h, pl.ds(s, 16)]
                obuf[h, pl.ds(16, 16)] = gbuf[h, pl.ds(s + 16, 16)]

    def start_write(j, slot):
        pltpu.async_copy(obufs[slot], out_hbm.at[base + j], wsem[slot])

    def wait_write(slot):
        pltpu.make_async_copy(obufs[slot], out_hbm.at[0], wsem[slot]).wait()

    for j in range(_LOOK):
        start_gather(j, j)

    for j in range(_LOOK):
        start_gather(j + _LOOK, j + _LOOK)
        wait_gather(j)
        extract(j, j)
        start_write(j, j)

    @pl.loop(_LOOK, _BPW - _LOOK, step=_NBUF)
    def _steady(g):
        for b in range(_NBUF):
            j = g + b
            s_ahead = (_LOOK + b + _LOOK) % _NBUF
            wait_write(s_ahead)
            start_gather(j + _LOOK, s_ahead)
            slot = (_LOOK + b) % _NBUF
            wait_gather(slot)
            extract(j, slot)
            start_write(j, slot)

    for t in range(_LOOK):
        j = _BPW - _LOOK + t
        slot = j % _NBUF
        wait_gather(slot)
        extract(j, slot)
        start_write(j, slot)

    for b in range(_NBUF):
        wait_write(b)


def kernel(inputs, context_weight):
    idx = inputs.astype(jnp.int32)
    ridx = jnp.pad(idx >> 2, ((0, 0), (0, _HPAD - _HIST))).reshape(-1)
    seg = jnp.pad((idx & 3) << 5, ((0, 0), (0, _HPAD - _HIST))).reshape(-1)
    table128 = context_weight.reshape(-1, _PROW)
    return _gather_kernel(ridx, seg, table128)
